# Initial kernel scaffold; baseline (speedup 1.0000x reference)
import jax
import jax.numpy as jnp
from jax.experimental import pallas as pl


def kernel(x, conv_w, centroids):
    raise NotImplementedError("write your pallas kernel here")


# speedup vs baseline: 1.5903x; 1.5903x over previous
import jax
import jax.numpy as jnp
from jax.experimental import pallas as pl
from jax.experimental.pallas import tpu as pltpu

_K = 64
_C = 2048


def _netvlad_kernel(x_ref, w_ref, c_ref, o_ref):
    xb = x_ref[0]
    w = w_ref[...]
    logits = jnp.dot(w, xb, preferred_element_type=jnp.float32)
    m = jnp.max(logits, axis=0, keepdims=True)
    e = jnp.exp(logits - m)
    s = jnp.sum(e, axis=0, keepdims=True)
    a = e / s
    ax = jax.lax.dot_general(
        a, xb, (((1,), (1,)), ((), ())),
        preferred_element_type=jnp.float32)
    a_sum = jnp.sum(a, axis=1, keepdims=True)
    vlad = ax - a_sum * c_ref[...]
    nrm = jnp.sqrt(jnp.sum(vlad * vlad))
    inv = 1.0 / jnp.maximum(nrm, 1e-12)
    o_ref[0] = vlad * inv


def kernel(x, conv_w, centroids):
    B, C, N = x.shape
    K = conv_w.shape[0]
    out = pl.pallas_call(
        _netvlad_kernel,
        grid=(B,),
        in_specs=[
            pl.BlockSpec((1, C, N), lambda b: (b, 0, 0)),
            pl.BlockSpec((K, C), lambda b: (0, 0)),
            pl.BlockSpec((K, C), lambda b: (0, 0)),
        ],
        out_specs=pl.BlockSpec((1, K, C), lambda b: (b, 0, 0)),
        out_shape=jax.ShapeDtypeStruct((B, K, C), jnp.float32),
        compiler_params=pltpu.CompilerParams(
            dimension_semantics=("parallel",),
        ),
    )(x, conv_w, centroids)
    return out.reshape(B, K * C)


# speedup vs baseline: 1.6090x; 1.0118x over previous
import jax
import jax.numpy as jnp
from jax.experimental import pallas as pl
from jax.experimental.pallas import tpu as pltpu

_NS = 2


def _netvlad_kernel(*refs):
    x_refs = refs[:_NS]
    w_ref, c_ref, o_ref = refs[_NS:]
    K, C = w_ref.shape
    Cs = C // _NS
    logits = jnp.dot(w_ref[:, 0:Cs], x_refs[0][0],
                     preferred_element_type=jnp.float32)
    for j in range(1, _NS):
        logits = logits + jnp.dot(w_ref[:, j * Cs:(j + 1) * Cs],
                                  x_refs[j][0],
                                  preferred_element_type=jnp.float32)
    m = jnp.max(logits, axis=0, keepdims=True)
    e = jnp.exp(logits - m)
    s = jnp.sum(e, axis=0, keepdims=True)
    a = e / s
    a_sum = jnp.sum(a, axis=1, keepdims=True)
    vlads = []
    sq = 0.0
    for j in range(_NS):
        ax = jax.lax.dot_general(
            a, x_refs[j][0], (((1,), (1,)), ((), ())),
            preferred_element_type=jnp.float32)
        vlad = ax - a_sum * c_ref[:, j * Cs:(j + 1) * Cs]
        vlads.append(vlad)
        sq = sq + jnp.sum(vlad * vlad)
    inv = 1.0 / jnp.maximum(jnp.sqrt(sq), 1e-12)
    for j in range(_NS):
        o_ref[0, :, j * Cs:(j + 1) * Cs] = vlads[j] * inv


def kernel(x, conv_w, centroids):
    B, C, N = x.shape
    K = conv_w.shape[0]
    Cs = C // _NS
    x_specs = [
        pl.BlockSpec((1, Cs, N), lambda b, j=j: (b, j, 0)) for j in range(_NS)
    ]
    out = pl.pallas_call(
        _netvlad_kernel,
        grid=(B,),
        in_specs=x_specs + [
            pl.BlockSpec((K, C), lambda b: (0, 0)),
            pl.BlockSpec((K, C), lambda b: (0, 0)),
        ],
        out_specs=pl.BlockSpec((1, K, C), lambda b: (b, 0, 0)),
        out_shape=jax.ShapeDtypeStruct((B, K, C), jnp.float32),
        compiler_params=pltpu.CompilerParams(
            dimension_semantics=("arbitrary",),
        ),
    )(*([x] * _NS), conv_w, centroids)
    return out.reshape(B, K * C)
